# baseline (speedup 1.0000x reference)
import functools

import jax
import jax.numpy as jnp
from jax import lax
from jax.experimental import pallas as pl
from jax.experimental.pallas import tpu as pltpu
from jax.experimental.pallas import tpu_sc as plsc

NC = 2
NS = 16
NW = NC * NS

_SC_PARAMS = pltpu.CompilerParams(needs_layout_passes=False,
                                  use_tc_tiling_on_sc=False)


def _leaky(v):
    return jnp.maximum(v, 0.001 * v)


def _precompute(emb, W0, b0, angle_table):
    n, d = emb.shape
    nt = angle_table.shape[0]
    bn = 2000

    def body(emb_ref, w0_ref, b0_ref, tab_ref, a4_ref, b_ref, c_ref):
        e = emb_ref[...]
        a = jnp.dot(e, w0_ref[0:128, :], preferred_element_type=jnp.float32)
        t = jnp.dot(tab_ref[...], w0_ref[384:448, :],
                    preferred_element_type=jnp.float32) + b0_ref[...]
        a4_ref[...] = (a[:, None, :] + t[None, :, :]).reshape(bn * nt, 128).astype(jnp.bfloat16)
        b_ref[...] = jnp.dot(e, w0_ref[128:256, :], preferred_element_type=jnp.float32).astype(jnp.bfloat16)
        c_ref[...] = jnp.dot(e, w0_ref[256:384, :], preferred_element_type=jnp.float32).astype(jnp.bfloat16)

    return pl.pallas_call(
        body,
        grid=(n // bn,),
        in_specs=[
            pl.BlockSpec((bn, d), lambda i: (i, 0)),
            pl.BlockSpec((448, 128), lambda i: (0, 0)),
            pl.BlockSpec((1, 128), lambda i: (0, 0)),
            pl.BlockSpec((nt, 64), lambda i: (0, 0)),
        ],
        out_specs=[
            pl.BlockSpec((bn * nt, 128), lambda i: (i, 0)),
            pl.BlockSpec((bn, 128), lambda i: (i, 0)),
            pl.BlockSpec((bn, 128), lambda i: (i, 0)),
        ],
        out_shape=[
            jax.ShapeDtypeStruct((n * nt, 128), jnp.bfloat16),
            jax.ShapeDtypeStruct((n, 128), jnp.bfloat16),
            jax.ShapeDtypeStruct((n, 128), jnp.bfloat16),
        ],
    )(emb, W0, b0, angle_table)


def _sc_embed(A, B, C, i0r, i1r, i2r):
    nchunks = i0r.shape[1]
    K = i0r.shape[2]
    epw = nchunks * K
    e = NW * epw
    mesh = plsc.VectorSubcoreMesh(core_axis_name="c", subcore_axis_name="s",
                                  num_cores=NC, num_subcores=NS)

    @functools.partial(
        pl.kernel,
        out_type=jax.ShapeDtypeStruct((e, 128), jnp.bfloat16),
        mesh=mesh,
        compiler_params=_SC_PARAMS,
        scratch_types=[
            pltpu.VMEM((nchunks, K), jnp.int32),
            pltpu.VMEM((nchunks, K), jnp.int32),
            pltpu.VMEM((nchunks, K), jnp.int32),
        ]
        + [pltpu.VMEM((K, 128), jnp.bfloat16)] * 6
        + [pltpu.VMEM((K, 128), jnp.bfloat16)] * 2
        + [pltpu.SemaphoreType.DMA] * 4,
    )
    def k(a_hbm, b_hbm, c_hbm, i0_hbm, i1_hbm, i2_hbm, s_hbm,
          iv0, iv1, iv2, ra0, rb0, rc0, ra1, rb1, rc1, sb0, sb1,
          gsem0, gsem1, ssem0, ssem1):
        wid = lax.axis_index("s") * NC + lax.axis_index("c")
        base = wid * epw
        pltpu.sync_copy(i0_hbm.at[wid], iv0)
        pltpu.sync_copy(i1_hbm.at[wid], iv1)
        pltpu.sync_copy(i2_hbm.at[wid], iv2)
        sets = ((ra0, rb0, rc0, sb0, gsem0, ssem0),
                (ra1, rb1, rc1, sb1, gsem1, ssem1))

        def issue(ci, st):
            ra, rb, rc, _, gsem, _ = st
            pltpu.async_copy(a_hbm.at[iv0.at[ci]], ra, gsem)
            pltpu.async_copy(b_hbm.at[iv1.at[ci]], rb, gsem)
            pltpu.async_copy(c_hbm.at[iv2.at[ci]], rc, gsem)

        def drain_gathers(st):
            ra, _, _, _, gsem, _ = st
            pltpu.make_async_copy(a_hbm.at[iv0.at[0]], ra, gsem).wait()
            pltpu.make_async_copy(a_hbm.at[iv0.at[0]], ra, gsem).wait()
            pltpu.make_async_copy(a_hbm.at[iv0.at[0]], ra, gsem).wait()

        def process(ci, st, first):
            ra, rb, rc, sb, _, ssem = st
            drain_gathers(st)
            if not first:
                pltpu.make_async_copy(sb, s_hbm.at[pl.ds(0, K), :], ssem).wait()

            def row(j, c2):
                for l in range(4):
                    sl = pl.ds(l * 32, 32)
                    sb[j, sl] = ra[j, sl] + rb[j, sl] + rc[j, sl]
                return c2

            lax.fori_loop(0, K, row, 0)
            pltpu.async_copy(sb, s_hbm.at[pl.ds(base + ci * K, K), :], ssem)

        issue(0, sets[0])
        issue(1, sets[1])
        process(0, sets[0], True)
        issue(2, sets[0])
        process(1, sets[1], True)
        issue(3, sets[1])

        def pair_body(j, carry):
            c0 = 2 + j * 2
            process(c0, sets[0], False)

            @pl.when(c0 + 2 < nchunks)
            def _():
                issue(c0 + 2, sets[0])

            @pl.when(c0 + 1 < nchunks)
            def _():
                process(c0 + 1, sets[1], False)

                @pl.when(c0 + 3 < nchunks)
                def _():
                    issue(c0 + 3, sets[1])

            return carry

        lax.fori_loop(0, (nchunks - 2 + 1) // 2, pair_body, 0)
        pltpu.make_async_copy(sb0, s_hbm.at[pl.ds(0, K), :], ssem0).wait()
        pltpu.make_async_copy(sb1, s_hbm.at[pl.ds(0, K), :], ssem1).wait()

    return k(A, B, C, i0r, i1r, i2r)


def _sc_geom(xx, xy, xz, px, py, pz, i0, i1, i2):
    n = xx.shape[0]
    e = i0.shape[0]
    epw = e // NW
    CH = 2000
    G = CH // 16
    nouter = epw // CH
    mesh = plsc.VectorSubcoreMesh(core_axis_name="c", subcore_axis_name="s",
                                  num_cores=NC, num_subcores=NS)
    fvec = jax.ShapeDtypeStruct((e,), jnp.float32)

    @functools.partial(
        pl.kernel,
        out_type=(fvec,) * 15,
        mesh=mesh,
        compiler_params=_SC_PARAMS,
        scratch_types=[pltpu.VMEM((n,), jnp.float32)] * 6
        + [pltpu.VMEM((CH,), jnp.int32)] * 3
        + [pltpu.VMEM((CH,), jnp.float32)] * 15,
    )
    def k(xx_h, xy_h, xz_h, px_h, py_h, pz_h, i0_h, i1_h, i2_h,
          r0x_h, r0y_h, r0z_h, r2x_h, r2y_h, r2z_h,
          q0x_h, q0y_h, q0z_h, q2x_h, q2y_h, q2z_h, p1x_h, p1y_h, p1z_h,
          xx_v, xy_v, xz_v, px_v, py_v, pz_v, iv0, iv1, iv2,
          r0x, r0y, r0z, r2x, r2y, r2z,
          q0x, q0y, q0z, q2x, q2y, q2z, p1x, p1y, p1z):
        wid = lax.axis_index("s") * NC + lax.axis_index("c")
        base = wid * epw
        pltpu.sync_copy(xx_h, xx_v)
        pltpu.sync_copy(xy_h, xy_v)
        pltpu.sync_copy(xz_h, xz_v)
        pltpu.sync_copy(px_h, px_v)
        pltpu.sync_copy(py_h, py_v)
        pltpu.sync_copy(pz_h, pz_v)

        def outer(oi, carry):
            off = base + oi * CH
            pltpu.sync_copy(i0_h.at[pl.ds(off, CH)], iv0)
            pltpu.sync_copy(i1_h.at[pl.ds(off, CH)], iv1)
            pltpu.sync_copy(i2_h.at[pl.ds(off, CH)], iv2)

            def grp(g, c2):
                sl = pl.ds(g * 16, 16)
                a0 = iv0[sl]
                a1 = iv1[sl]
                a2 = iv2[sl]
                x1 = plsc.load_gather(xx_v, [a1])
                y1 = plsc.load_gather(xy_v, [a1])
                z1 = plsc.load_gather(xz_v, [a1])
                r0x[sl] = plsc.load_gather(xx_v, [a0]) - x1
                r0y[sl] = plsc.load_gather(xy_v, [a0]) - y1
                r0z[sl] = plsc.load_gather(xz_v, [a0]) - z1
                r2x[sl] = plsc.load_gather(xx_v, [a2]) - x1
                r2y[sl] = plsc.load_gather(xy_v, [a2]) - y1
                r2z[sl] = plsc.load_gather(xz_v, [a2]) - z1
                u1 = plsc.load_gather(px_v, [a1])
                v1 = plsc.load_gather(py_v, [a1])
                w1 = plsc.load_gather(pz_v, [a1])
                q0x[sl] = plsc.load_gather(px_v, [a0]) - u1
                q0y[sl] = plsc.load_gather(py_v, [a0]) - v1
                q0z[sl] = plsc.load_gather(pz_v, [a0]) - w1
                q2x[sl] = plsc.load_gather(px_v, [a2]) - u1
                q2y[sl] = plsc.load_gather(py_v, [a2]) - v1
                q2z[sl] = plsc.load_gather(pz_v, [a2]) - w1
                p1x[sl] = u1
                p1y[sl] = v1
                p1z[sl] = w1
                return c2

            lax.fori_loop(0, G, grp, 0)
            dst = pl.ds(off, CH)
            pltpu.sync_copy(r0x, r0x_h.at[dst])
            pltpu.sync_copy(r0y, r0y_h.at[dst])
            pltpu.sync_copy(r0z, r0z_h.at[dst])
            pltpu.sync_copy(r2x, r2x_h.at[dst])
            pltpu.sync_copy(r2y, r2y_h.at[dst])
            pltpu.sync_copy(r2z, r2z_h.at[dst])
            pltpu.sync_copy(q0x, q0x_h.at[dst])
            pltpu.sync_copy(q0y, q0y_h.at[dst])
            pltpu.sync_copy(q0z, q0z_h.at[dst])
            pltpu.sync_copy(q2x, q2x_h.at[dst])
            pltpu.sync_copy(q2y, q2y_h.at[dst])
            pltpu.sync_copy(q2z, q2z_h.at[dst])
            pltpu.sync_copy(p1x, p1x_h.at[dst])
            pltpu.sync_copy(p1y, p1y_h.at[dst])
            pltpu.sync_copy(p1z, p1z_h.at[dst])
            return carry

        lax.fori_loop(0, nouter, outer, 0)

    return k(xx, xy, xz, px, py, pz, i0, i1, i2)


def _mlp(s, W1, b1, g1, be1, W2, b2, g2, be2, W3T, b3p):
    e = s.shape[0]
    BE = 2560

    def body(s_ref, w1_ref, b1_ref, g1_ref, be1_ref,
             w2_ref, b2_ref, g2_ref, be2_ref, w3_ref, b3_ref, th0_ref, th2_ref):
        def ln_leaky(h, g, be):
            m = jnp.mean(h, axis=-1, keepdims=True)
            msq = jnp.mean(h * h, axis=-1, keepdims=True)
            rstd = lax.rsqrt(jnp.maximum(msq - m * m, 0.0) + 1e-5)
            return _leaky((h - m) * rstd * g + be)

        h = _leaky(s_ref[...].astype(jnp.float32))
        h = jnp.dot(h, w1_ref[...], preferred_element_type=jnp.float32) + b1_ref[...]
        h = ln_leaky(h, g1_ref[...], be1_ref[...])
        h = jnp.dot(h, w2_ref[...], preferred_element_type=jnp.float32) + b2_ref[...]
        h = ln_leaky(h, g2_ref[...], be2_ref[...])
        f01 = lax.dot_general(w3_ref[...], h, (((1,), (1,)), ((), ())),
                              preferred_element_type=jnp.float32)
        off = pl.program_id(0) * BE
        th0_ref[pl.ds(off, BE)] = (f01[0] + b3_ref[0:1, 0:1][0, 0]) * 0.5
        th2_ref[pl.ds(off, BE)] = (f01[1] + b3_ref[0:1, 1:2][0, 0]) * 0.5

    full = lambda shape: pl.BlockSpec(shape, lambda i: (0, 0))
    return pl.pallas_call(
        body,
        grid=(e // BE,),
        in_specs=[
            pl.BlockSpec((BE, 128), lambda i: (i, 0)),
            full((128, 128)), full((1, 128)),
            full((1, 128)), full((1, 128)), full((128, 128)), full((1, 128)),
            full((1, 128)), full((1, 128)), full((2, 128)), full((1, 128)),
        ],
        out_specs=[
            pl.BlockSpec((e,), lambda i: (0,)),
            pl.BlockSpec((e,), lambda i: (0,)),
        ],
        out_shape=[
            jax.ShapeDtypeStruct((e,), jnp.float32),
            jax.ShapeDtypeStruct((e,), jnp.float32),
        ],
    )(s, W1, b1, g1, be1, W2, b2, g2, be2, W3T, b3p)


def _rotate(th0, th2, r0x, r0y, r0z, r2x, r2y, r2z,
            q0x, q0y, q0z, q2x, q2y, q2z, p1x, p1y, p1z):
    e = th0.shape[0]
    BE = 2560

    def body(th0_ref, th2_ref, ax_ref, ay_ref, az_ref, bx_ref, by_ref, bz_ref,
             q0x_ref, q0y_ref, q0z_ref, q2x_ref, q2y_ref, q2z_ref,
             p1x_ref, p1y_ref, p1z_ref,
             o0x_ref, o0y_ref, o0z_ref, o2x_ref, o2y_ref, o2z_ref):
        sl = pl.ds(pl.program_id(0) * BE, BE)
        ax, ay, az = ax_ref[sl], ay_ref[sl], az_ref[sl]
        bx, by, bz = bx_ref[sl], by_ref[sl], bz_ref[sl]
        ia = 1.0 / (jnp.sqrt(ax * ax + ay * ay + az * az) + 1e-12)
        ib = 1.0 / (jnp.sqrt(bx * bx + by * by + bz * bz) + 1e-12)
        d1x, d1y, d1z = ax * ia, ay * ia, az * ia
        d2x, d2y, d2z = bx * ib, by * ib, bz * ib
        nx = d1y * d2z - d1z * d2y
        ny = d1z * d2x - d1x * d2z
        nz = d1x * d2y - d1y * d2x
        inn = 1.0 / (jnp.sqrt(nx * nx + ny * ny + nz * nz) + 1e-12)
        nx, ny, nz = nx * inn, ny * inn, nz * inn

        def rot(theta, ux, uy, uz, vx, vy, vz):
            c = jnp.cos(theta)
            s = jnp.sin(theta)
            t = 1.0 - c
            rx = (c + t * ux * ux) * vx + (t * ux * uy - s * uz) * vy + (t * ux * uz + s * uy) * vz
            ry = (t * ux * uy + s * uz) * vx + (c + t * uy * uy) * vy + (t * uy * uz - s * ux) * vz
            rz = (t * ux * uz - s * uy) * vx + (t * uy * uz + s * ux) * vy + (c + t * uz * uz) * vz
            return rx, ry, rz

        x0, y0, z0 = rot(th0_ref[sl], nx, ny, nz,
                         q0x_ref[sl], q0y_ref[sl], q0z_ref[sl])
        x2, y2, z2 = rot(th2_ref[sl], -nx, -ny, -nz,
                         q2x_ref[sl], q2y_ref[sl], q2z_ref[sl])
        o0x_ref[sl] = x0 + p1x_ref[sl]
        o0y_ref[sl] = y0 + p1y_ref[sl]
        o0z_ref[sl] = z0 + p1z_ref[sl]
        o2x_ref[sl] = x2 + p1x_ref[sl]
        o2y_ref[sl] = y2 + p1y_ref[sl]
        o2z_ref[sl] = z2 + p1z_ref[sl]

    blk = pl.BlockSpec((e,), lambda i: (0,))
    fmat = jax.ShapeDtypeStruct((e,), jnp.float32)
    return pl.pallas_call(
        body,
        grid=(e // BE,),
        in_specs=[blk] * 17,
        out_specs=[blk] * 6,
        out_shape=[fmat] * 6,
    )(th0, th2, r0x, r0y, r0z, r2x, r2y, r2z,
      q0x, q0y, q0z, q2x, q2y, q2z, p1x, p1y, p1z)


def _sc_scatter(n, o0x, o0y, o0z, p1x, p1y, p1z, o2x, o2y, o2z, i0r, i1r, i2r):
    nchunks = i0r.shape[1]
    K = i0r.shape[2]
    epw = nchunks * K
    rows_per_tile = n // NS
    mesh = plsc.VectorSubcoreMesh(core_axis_name="c", subcore_axis_name="s",
                                  num_cores=NC, num_subcores=NS)

    @functools.partial(
        pl.kernel,
        out_type=jax.ShapeDtypeStruct((NC, n, 16), jnp.float32),
        mesh=mesh,
        compiler_params=_SC_PARAMS,
        scratch_types=[
            pltpu.VMEM_SHARED((n, 16), jnp.float32),
            pltpu.VMEM((rows_per_tile, 16), jnp.float32),
        ]
        + [pltpu.VMEM((K,), jnp.int32)] * 2
        + [pltpu.VMEM((K,), jnp.float32)] * 6
        + [pltpu.VMEM((K, 16), jnp.float32)] * 2
        + [pltpu.SemaphoreType.DMA] * 2,
    )
    def k(o0x_h, o0y_h, o0z_h, p1x_h, p1y_h, p1z_h, o2x_h, o2y_h, o2z_h,
          i0_h, i1_h, i2_h, out_h,
          acc, stage, iv0, iv1, vx0, vy0, vz0, vx1, vy1, vz1,
          valbuf0, valbuf1, lsem0, lsem1):
        cid = lax.axis_index("c")
        sid = lax.axis_index("s")
        wid = sid * NC + cid
        zero16 = jnp.zeros((16,), jnp.float32)
        one16 = jnp.ones((16,), jnp.float32)
        lane = lax.iota(jnp.int32, 16)
        sets = ((iv0, vx0, vy0, vz0, valbuf0, lsem0),
                (iv1, vx1, vy1, vz1, valbuf1, lsem1))

        def z1(j, c2):
            stage[j, :] = zero16
            return c2

        lax.fori_loop(0, rows_per_tile, z1, 0)
        pltpu.sync_copy(stage, acc.at[pl.ds(sid * rows_per_tile, rows_per_tile), :])

        def z2(j, c2):
            valbuf0[j, :] = zero16
            valbuf1[j, :] = zero16
            return c2

        lax.fori_loop(0, K, z2, 0)
        plsc.subcore_barrier()

        def make_pass(idx_h, cx_h, cy_h, cz_h):
            def issue(ci, st):
                iv, vx, vy, vz, _, lsem = st
                pltpu.async_copy(idx_h.at[wid, ci], iv, lsem)
                pltpu.async_copy(cx_h.at[pl.ds(wid * epw + ci * K, K)], vx, lsem)
                pltpu.async_copy(cy_h.at[pl.ds(wid * epw + ci * K, K)], vy, lsem)
                pltpu.async_copy(cz_h.at[pl.ds(wid * epw + ci * K, K)], vz, lsem)

            def process(ci, st):
                iv, vx, vy, vz, valbuf, lsem = st
                for _ in range(3):
                    pltpu.make_async_copy(cx_h.at[pl.ds(0, K)], vx, lsem).wait()
                pltpu.make_async_copy(idx_h.at[wid, 0], iv, lsem).wait()

                def grp(g, c2):
                    sl = pl.ds(g * 16, 16)
                    row = lane + g * 16
                    plsc.store_scatter(valbuf, [row, lane * 0], vx[sl])
                    plsc.store_scatter(valbuf, [row, lane * 0 + 1], vy[sl])
                    plsc.store_scatter(valbuf, [row, lane * 0 + 2], vz[sl])
                    plsc.store_scatter(valbuf, [row, lane * 0 + 3], one16)
                    return c2

                lax.fori_loop(0, K // 16, grp, 0)
                pltpu.sync_copy(valbuf, acc.at[iv], add=True)

            issue(0, sets[0])
            issue(1, sets[1])

            def pair_body(j, carry):
                c0 = j * 2
                process(c0, sets[0])

                @pl.when(c0 + 2 < nchunks)
                def _():
                    issue(c0 + 2, sets[0])

                process(c0 + 1, sets[1])

                @pl.when(c0 + 3 < nchunks)
                def _():
                    issue(c0 + 3, sets[1])

                return carry

            lax.fori_loop(0, nchunks // 2, pair_body, 0)
            if nchunks % 2:
                process(nchunks - 1, sets[0])

        make_pass(i0_h, o0x_h, o0y_h, o0z_h)
        make_pass(i1_h, p1x_h, p1y_h, p1z_h)
        make_pass(i2_h, o2x_h, o2y_h, o2z_h)
        plsc.subcore_barrier()
        rsl = pl.ds(sid * rows_per_tile, rows_per_tile)
        pltpu.sync_copy(acc.at[rsl, :], stage)
        pltpu.sync_copy(stage, out_h.at[cid, rsl, :])

    return k(o0x, o0y, o0z, p1x, p1y, p1z, o2x, o2y, o2z, i0r, i1r, i2r)


def _finalize(P, pos16):
    n = pos16.shape[0]

    def body(p_ref, pos_ref, out_ref):
        num = p_ref[0] + p_ref[1]
        cnt = num[:, 3:4]
        mean = num / jnp.maximum(cnt, 1.0)
        out_ref[...] = jnp.where(cnt > 0, mean, pos_ref[...])

    return pl.pallas_call(
        body,
        in_specs=[
            pl.BlockSpec((2, n, 16), lambda: (0, 0, 0)),
            pl.BlockSpec((n, 16), lambda: (0, 0)),
        ],
        out_specs=pl.BlockSpec((n, 16), lambda: (0, 0)),
        out_shape=jax.ShapeDtypeStruct((n, 16), jnp.float32),
    )(P, pos16)


def kernel(x, angle_index, node_embedding, pos, angle_types,
           W0, b0, W1, b1, g1, be1, W2, b2, g2, be2, W3, b3, angle_table):
    n = x.shape[0]
    e = angle_index.shape[0]

    i0 = angle_index[:, 0].astype(jnp.int32)
    i1 = angle_index[:, 1].astype(jnp.int32)
    i2 = angle_index[:, 2].astype(jnp.int32)
    xx, xy, xz = x[:, 0], x[:, 1], x[:, 2]
    px, py, pz = pos[:, 0], pos[:, 1], pos[:, 2]

    K = 80
    nchunks = e // (NW * K)
    chunked = lambda a: a.reshape(NW, nchunks, K)
    nt = angle_table.shape[0]
    tt = angle_types.astype(jnp.int32)
    j0 = i0 * nt + tt
    j0r, i1r, i2r = chunked(j0), chunked(i1), chunked(i2)
    i0r = chunked(i0)

    A4, B, C = _precompute(node_embedding, W0, b0[None, :], angle_table)
    s = _sc_embed(A4, B, C, j0r, i1r, i2r)
    (r0x, r0y, r0z, r2x, r2y, r2z,
     q0x, q0y, q0z, q2x, q2y, q2z, p1x, p1y, p1z) = _sc_geom(
        xx, xy, xz, px, py, pz, i0, i1, i2)

    b3p = jnp.concatenate([b3, jnp.zeros((126,), jnp.float32)])[None, :]
    th0, th2 = _mlp(s, W1, b1[None, :], g1[None, :],
                    be1[None, :], W2, b2[None, :], g2[None, :], be2[None, :],
                    W3.T, b3p)

    o0x, o0y, o0z, o2x, o2y, o2z = _rotate(
        th0, th2,
        r0x, r0y, r0z, r2x, r2y, r2z,
        q0x, q0y, q0z, q2x, q2y, q2z, p1x, p1y, p1z)

    npad = ((n + 127) // 128) * 128
    P = _sc_scatter(npad, o0x, o0y, o0z, p1x, p1y, p1z,
                    o2x, o2y, o2z, i0r, i1r, i2r)

    pos16 = jnp.concatenate(
        [pos, jnp.zeros((n, 13), jnp.float32)], axis=1)
    pos16 = jnp.concatenate(
        [pos16, jnp.zeros((npad - n, 16), jnp.float32)], axis=0)
    out16 = _finalize(P, pos16)
    return out16[:n, :3]


# speedup vs baseline: 1.4468x; 1.4468x over previous
import functools

import jax
import jax.numpy as jnp
from jax import lax
from jax.experimental import pallas as pl
from jax.experimental.pallas import tpu as pltpu
from jax.experimental.pallas import tpu_sc as plsc

NC = 2
NS = 16
NW = NC * NS

_SC_PARAMS = pltpu.CompilerParams(needs_layout_passes=False,
                                  use_tc_tiling_on_sc=False)


def _leaky(v):
    return jnp.maximum(v, 0.001 * v)


def _precompute(emb, W0, b0, angle_table):
    n, d = emb.shape
    nt = angle_table.shape[0]
    bn = 2000

    def body(emb_ref, w0_ref, b0_ref, tab_ref, a4_ref, b_ref, c_ref):
        e = emb_ref[...]
        a = jnp.dot(e, w0_ref[0:128, :], preferred_element_type=jnp.float32)
        t = jnp.dot(tab_ref[...], w0_ref[384:448, :],
                    preferred_element_type=jnp.float32) + b0_ref[...]
        a4_ref[...] = (a[:, None, :] + t[None, :, :]).reshape(bn * nt, 128)
        b_ref[...] = jnp.dot(e, w0_ref[128:256, :], preferred_element_type=jnp.float32)
        c_ref[...] = jnp.dot(e, w0_ref[256:384, :], preferred_element_type=jnp.float32)

    return pl.pallas_call(
        body,
        grid=(n // bn,),
        in_specs=[
            pl.BlockSpec((bn, d), lambda i: (i, 0)),
            pl.BlockSpec((448, 128), lambda i: (0, 0)),
            pl.BlockSpec((1, 128), lambda i: (0, 0)),
            pl.BlockSpec((nt, 64), lambda i: (0, 0)),
        ],
        out_specs=[
            pl.BlockSpec((bn * nt, 128), lambda i: (i, 0)),
            pl.BlockSpec((bn, 128), lambda i: (i, 0)),
            pl.BlockSpec((bn, 128), lambda i: (i, 0)),
        ],
        out_shape=[
            jax.ShapeDtypeStruct((n * nt, 128), jnp.float32),
            jax.ShapeDtypeStruct((n, 128), jnp.float32),
            jax.ShapeDtypeStruct((n, 128), jnp.float32),
        ],
    )(emb, W0, b0, angle_table)


def _sc_embed(A, B, C, i0r, i1r, i2r):
    nchunks = i0r.shape[1]
    K = i0r.shape[2]
    epw = nchunks * K
    e = NW * epw
    mesh = plsc.VectorSubcoreMesh(core_axis_name="c", subcore_axis_name="s",
                                  num_cores=NC, num_subcores=NS)

    @functools.partial(
        pl.kernel,
        out_type=jax.ShapeDtypeStruct((e, 128), jnp.float32),
        mesh=mesh,
        compiler_params=_SC_PARAMS,
        scratch_types=[
            pltpu.VMEM((nchunks, K), jnp.int32),
            pltpu.VMEM((nchunks, K), jnp.int32),
            pltpu.VMEM((nchunks, K), jnp.int32),
        ]
        + [pltpu.VMEM((K, 128), jnp.float32)] * 6
        + [pltpu.VMEM((K, 128), jnp.float32)] * 2
        + [pltpu.SemaphoreType.DMA] * 4,
    )
    def k(a_hbm, b_hbm, c_hbm, i0_hbm, i1_hbm, i2_hbm, s_hbm,
          iv0, iv1, iv2, ra0, rb0, rc0, ra1, rb1, rc1, sb0, sb1,
          gsem0, gsem1, ssem0, ssem1):
        wid = lax.axis_index("s") * NC + lax.axis_index("c")
        base = wid * epw
        pltpu.sync_copy(i0_hbm.at[wid], iv0)
        pltpu.sync_copy(i1_hbm.at[wid], iv1)
        pltpu.sync_copy(i2_hbm.at[wid], iv2)
        sets = ((ra0, rb0, rc0, sb0, gsem0, ssem0),
                (ra1, rb1, rc1, sb1, gsem1, ssem1))

        def issue(ci, st):
            ra, rb, rc, _, gsem, _ = st
            pltpu.async_copy(a_hbm.at[iv0.at[ci]], ra, gsem)
            pltpu.async_copy(b_hbm.at[iv1.at[ci]], rb, gsem)
            pltpu.async_copy(c_hbm.at[iv2.at[ci]], rc, gsem)

        def drain_gathers(st):
            ra, _, _, _, gsem, _ = st
            pltpu.make_async_copy(a_hbm.at[iv0.at[0]], ra, gsem).wait()
            pltpu.make_async_copy(a_hbm.at[iv0.at[0]], ra, gsem).wait()
            pltpu.make_async_copy(a_hbm.at[iv0.at[0]], ra, gsem).wait()

        def process(ci, st, first):
            ra, rb, rc, sb, _, ssem = st
            drain_gathers(st)
            if not first:
                pltpu.make_async_copy(sb, s_hbm.at[pl.ds(0, K), :], ssem).wait()

            def row(j, c2):
                for l in range(8):
                    sl = pl.ds(l * 16, 16)
                    sb[j, sl] = ra[j, sl] + rb[j, sl] + rc[j, sl]
                return c2

            lax.fori_loop(0, K, row, 0)
            pltpu.async_copy(sb, s_hbm.at[pl.ds(base + ci * K, K), :], ssem)

        issue(0, sets[0])
        issue(1, sets[1])
        process(0, sets[0], True)
        issue(2, sets[0])
        process(1, sets[1], True)
        issue(3, sets[1])

        def pair_body(j, carry):
            c0 = 2 + j * 2
            process(c0, sets[0], False)

            @pl.when(c0 + 2 < nchunks)
            def _():
                issue(c0 + 2, sets[0])

            @pl.when(c0 + 1 < nchunks)
            def _():
                process(c0 + 1, sets[1], False)

                @pl.when(c0 + 3 < nchunks)
                def _():
                    issue(c0 + 3, sets[1])

            return carry

        lax.fori_loop(0, (nchunks - 2 + 1) // 2, pair_body, 0)
        pltpu.make_async_copy(sb0, s_hbm.at[pl.ds(0, K), :], ssem0).wait()
        pltpu.make_async_copy(sb1, s_hbm.at[pl.ds(0, K), :], ssem1).wait()

    return k(A, B, C, i0r, i1r, i2r)


def _sc_geom(xx, xy, xz, px, py, pz, i0, i1, i2):
    n = xx.shape[0]
    e = i0.shape[0]
    epw = e // NW
    CH = 2000
    G = CH // 16
    nouter = epw // CH
    mesh = plsc.VectorSubcoreMesh(core_axis_name="c", subcore_axis_name="s",
                                  num_cores=NC, num_subcores=NS)
    fvec = jax.ShapeDtypeStruct((e,), jnp.float32)

    @functools.partial(
        pl.kernel,
        out_type=(fvec,) * 15,
        mesh=mesh,
        compiler_params=_SC_PARAMS,
        scratch_types=[pltpu.VMEM((n,), jnp.float32)] * 6
        + [pltpu.VMEM((CH,), jnp.int32)] * 3
        + [pltpu.VMEM((CH,), jnp.float32)] * 15,
    )
    def k(xx_h, xy_h, xz_h, px_h, py_h, pz_h, i0_h, i1_h, i2_h,
          r0x_h, r0y_h, r0z_h, r2x_h, r2y_h, r2z_h,
          q0x_h, q0y_h, q0z_h, q2x_h, q2y_h, q2z_h, p1x_h, p1y_h, p1z_h,
          xx_v, xy_v, xz_v, px_v, py_v, pz_v, iv0, iv1, iv2,
          r0x, r0y, r0z, r2x, r2y, r2z,
          q0x, q0y, q0z, q2x, q2y, q2z, p1x, p1y, p1z):
        wid = lax.axis_index("s") * NC + lax.axis_index("c")
        base = wid * epw
        pltpu.sync_copy(xx_h, xx_v)
        pltpu.sync_copy(xy_h, xy_v)
        pltpu.sync_copy(xz_h, xz_v)
        pltpu.sync_copy(px_h, px_v)
        pltpu.sync_copy(py_h, py_v)
        pltpu.sync_copy(pz_h, pz_v)

        def outer(oi, carry):
            off = base + oi * CH
            pltpu.sync_copy(i0_h.at[pl.ds(off, CH)], iv0)
            pltpu.sync_copy(i1_h.at[pl.ds(off, CH)], iv1)
            pltpu.sync_copy(i2_h.at[pl.ds(off, CH)], iv2)

            def grp(g, c2):
                sl = pl.ds(g * 16, 16)
                a0 = iv0[sl]
                a1 = iv1[sl]
                a2 = iv2[sl]
                x1 = plsc.load_gather(xx_v, [a1])
                y1 = plsc.load_gather(xy_v, [a1])
                z1 = plsc.load_gather(xz_v, [a1])
                r0x[sl] = plsc.load_gather(xx_v, [a0]) - x1
                r0y[sl] = plsc.load_gather(xy_v, [a0]) - y1
                r0z[sl] = plsc.load_gather(xz_v, [a0]) - z1
                r2x[sl] = plsc.load_gather(xx_v, [a2]) - x1
                r2y[sl] = plsc.load_gather(xy_v, [a2]) - y1
                r2z[sl] = plsc.load_gather(xz_v, [a2]) - z1
                u1 = plsc.load_gather(px_v, [a1])
                v1 = plsc.load_gather(py_v, [a1])
                w1 = plsc.load_gather(pz_v, [a1])
                q0x[sl] = plsc.load_gather(px_v, [a0]) - u1
                q0y[sl] = plsc.load_gather(py_v, [a0]) - v1
                q0z[sl] = plsc.load_gather(pz_v, [a0]) - w1
                q2x[sl] = plsc.load_gather(px_v, [a2]) - u1
                q2y[sl] = plsc.load_gather(py_v, [a2]) - v1
                q2z[sl] = plsc.load_gather(pz_v, [a2]) - w1
                p1x[sl] = u1
                p1y[sl] = v1
                p1z[sl] = w1
                return c2

            lax.fori_loop(0, G, grp, 0)
            dst = pl.ds(off, CH)
            pltpu.sync_copy(r0x, r0x_h.at[dst])
            pltpu.sync_copy(r0y, r0y_h.at[dst])
            pltpu.sync_copy(r0z, r0z_h.at[dst])
            pltpu.sync_copy(r2x, r2x_h.at[dst])
            pltpu.sync_copy(r2y, r2y_h.at[dst])
            pltpu.sync_copy(r2z, r2z_h.at[dst])
            pltpu.sync_copy(q0x, q0x_h.at[dst])
            pltpu.sync_copy(q0y, q0y_h.at[dst])
            pltpu.sync_copy(q0z, q0z_h.at[dst])
            pltpu.sync_copy(q2x, q2x_h.at[dst])
            pltpu.sync_copy(q2y, q2y_h.at[dst])
            pltpu.sync_copy(q2z, q2z_h.at[dst])
            pltpu.sync_copy(p1x, p1x_h.at[dst])
            pltpu.sync_copy(p1y, p1y_h.at[dst])
            pltpu.sync_copy(p1z, p1z_h.at[dst])
            return carry

        lax.fori_loop(0, nouter, outer, 0)

    return k(xx, xy, xz, px, py, pz, i0, i1, i2)


def _mlp(s, W1, b1, g1, be1, W2, b2, g2, be2, W3T, b3p):
    e = s.shape[0]
    BE = 6400

    def body(s_ref, w1_ref, b1_ref, g1_ref, be1_ref,
             w2_ref, b2_ref, g2_ref, be2_ref, w3_ref, b3_ref, th0_ref, th2_ref):
        def ln_leaky(h, g, be):
            m = jnp.mean(h, axis=-1, keepdims=True)
            msq = jnp.mean(h * h, axis=-1, keepdims=True)
            rstd = lax.rsqrt(jnp.maximum(msq - m * m, 0.0) + 1e-5)
            return _leaky((h - m) * rstd * g + be)

        h = _leaky(s_ref[...])
        h = jnp.dot(h, w1_ref[...], preferred_element_type=jnp.float32) + b1_ref[...]
        h = ln_leaky(h, g1_ref[...], be1_ref[...])
        h = jnp.dot(h, w2_ref[...], preferred_element_type=jnp.float32) + b2_ref[...]
        h = ln_leaky(h, g2_ref[...], be2_ref[...])
        f01 = lax.dot_general(w3_ref[...], h, (((1,), (1,)), ((), ())),
                              preferred_element_type=jnp.float32)
        off = pl.program_id(0) * BE
        th0_ref[pl.ds(off, BE)] = (f01[0] + b3_ref[0:1, 0:1][0, 0]) * 0.5
        th2_ref[pl.ds(off, BE)] = (f01[1] + b3_ref[0:1, 1:2][0, 0]) * 0.5

    full = lambda shape: pl.BlockSpec(shape, lambda i: (0, 0))
    return pl.pallas_call(
        body,
        grid=(e // BE,),
        in_specs=[
            pl.BlockSpec((BE, 128), lambda i: (i, 0)),
            full((128, 128)), full((1, 128)),
            full((1, 128)), full((1, 128)), full((128, 128)), full((1, 128)),
            full((1, 128)), full((1, 128)), full((2, 128)), full((1, 128)),
        ],
        out_specs=[
            pl.BlockSpec((e,), lambda i: (0,)),
            pl.BlockSpec((e,), lambda i: (0,)),
        ],
        out_shape=[
            jax.ShapeDtypeStruct((e,), jnp.float32),
            jax.ShapeDtypeStruct((e,), jnp.float32),
        ],
    )(s, W1, b1, g1, be1, W2, b2, g2, be2, W3T, b3p)


def _rotate(th0, th2, r0x, r0y, r0z, r2x, r2y, r2z,
            q0x, q0y, q0z, q2x, q2y, q2z, p1x, p1y, p1z):
    e = th0.shape[0]
    BE = 2560

    def body(th0_ref, th2_ref, ax_ref, ay_ref, az_ref, bx_ref, by_ref, bz_ref,
             q0x_ref, q0y_ref, q0z_ref, q2x_ref, q2y_ref, q2z_ref,
             p1x_ref, p1y_ref, p1z_ref,
             o0x_ref, o0y_ref, o0z_ref, o2x_ref, o2y_ref, o2z_ref):
        sl = pl.ds(pl.program_id(0) * BE, BE)
        ax, ay, az = ax_ref[sl], ay_ref[sl], az_ref[sl]
        bx, by, bz = bx_ref[sl], by_ref[sl], bz_ref[sl]
        ia = 1.0 / (jnp.sqrt(ax * ax + ay * ay + az * az) + 1e-12)
        ib = 1.0 / (jnp.sqrt(bx * bx + by * by + bz * bz) + 1e-12)
        d1x, d1y, d1z = ax * ia, ay * ia, az * ia
        d2x, d2y, d2z = bx * ib, by * ib, bz * ib
        nx = d1y * d2z - d1z * d2y
        ny = d1z * d2x - d1x * d2z
        nz = d1x * d2y - d1y * d2x
        inn = 1.0 / (jnp.sqrt(nx * nx + ny * ny + nz * nz) + 1e-12)
        nx, ny, nz = nx * inn, ny * inn, nz * inn

        def rot(theta, ux, uy, uz, vx, vy, vz):
            c = jnp.cos(theta)
            s = jnp.sin(theta)
            t = 1.0 - c
            rx = (c + t * ux * ux) * vx + (t * ux * uy - s * uz) * vy + (t * ux * uz + s * uy) * vz
            ry = (t * ux * uy + s * uz) * vx + (c + t * uy * uy) * vy + (t * uy * uz - s * ux) * vz
            rz = (t * ux * uz - s * uy) * vx + (t * uy * uz + s * ux) * vy + (c + t * uz * uz) * vz
            return rx, ry, rz

        x0, y0, z0 = rot(th0_ref[sl], nx, ny, nz,
                         q0x_ref[sl], q0y_ref[sl], q0z_ref[sl])
        x2, y2, z2 = rot(th2_ref[sl], -nx, -ny, -nz,
                         q2x_ref[sl], q2y_ref[sl], q2z_ref[sl])
        o0x_ref[sl] = x0 + p1x_ref[sl]
        o0y_ref[sl] = y0 + p1y_ref[sl]
        o0z_ref[sl] = z0 + p1z_ref[sl]
        o2x_ref[sl] = x2 + p1x_ref[sl]
        o2y_ref[sl] = y2 + p1y_ref[sl]
        o2z_ref[sl] = z2 + p1z_ref[sl]

    blk = pl.BlockSpec((e,), lambda i: (0,))
    fmat = jax.ShapeDtypeStruct((e,), jnp.float32)
    return pl.pallas_call(
        body,
        grid=(e // BE,),
        in_specs=[blk] * 17,
        out_specs=[blk] * 6,
        out_shape=[fmat] * 6,
    )(th0, th2, r0x, r0y, r0z, r2x, r2y, r2z,
      q0x, q0y, q0z, q2x, q2y, q2z, p1x, p1y, p1z)


def _sc_scatter(n, o0x, o0y, o0z, p1x, p1y, p1z, o2x, o2y, o2z, i0r, i1r, i2r):
    nchunks = i0r.shape[1]
    K = i0r.shape[2]
    epw = nchunks * K
    rows_per_tile = n // NS
    mesh = plsc.VectorSubcoreMesh(core_axis_name="c", subcore_axis_name="s",
                                  num_cores=NC, num_subcores=NS)

    @functools.partial(
        pl.kernel,
        out_type=jax.ShapeDtypeStruct((NC, n, 16), jnp.float32),
        mesh=mesh,
        compiler_params=_SC_PARAMS,
        scratch_types=[
            pltpu.VMEM_SHARED((n, 16), jnp.float32),
            pltpu.VMEM((rows_per_tile, 16), jnp.float32),
        ]
        + [pltpu.VMEM((K,), jnp.int32)] * 2
        + [pltpu.VMEM((K,), jnp.float32)] * 6
        + [pltpu.VMEM((K, 16), jnp.float32)] * 2
        + [pltpu.SemaphoreType.DMA] * 2,
    )
    def k(o0x_h, o0y_h, o0z_h, p1x_h, p1y_h, p1z_h, o2x_h, o2y_h, o2z_h,
          i0_h, i1_h, i2_h, out_h,
          acc, stage, iv0, iv1, vx0, vy0, vz0, vx1, vy1, vz1,
          valbuf0, valbuf1, lsem0, lsem1):
        cid = lax.axis_index("c")
        sid = lax.axis_index("s")
        wid = sid * NC + cid
        zero16 = jnp.zeros((16,), jnp.float32)
        one16 = jnp.ones((16,), jnp.float32)
        lane = lax.iota(jnp.int32, 16)
        sets = ((iv0, vx0, vy0, vz0, valbuf0, lsem0),
                (iv1, vx1, vy1, vz1, valbuf1, lsem1))

        def z1(j, c2):
            stage[j, :] = zero16
            return c2

        lax.fori_loop(0, rows_per_tile, z1, 0)
        pltpu.sync_copy(stage, acc.at[pl.ds(sid * rows_per_tile, rows_per_tile), :])

        def z2(j, c2):
            valbuf0[j, :] = zero16
            valbuf1[j, :] = zero16
            return c2

        lax.fori_loop(0, K, z2, 0)
        plsc.subcore_barrier()

        def make_pass(idx_h, cx_h, cy_h, cz_h):
            def issue(ci, st):
                iv, vx, vy, vz, _, lsem = st
                pltpu.async_copy(idx_h.at[wid, ci], iv, lsem)
                pltpu.async_copy(cx_h.at[pl.ds(wid * epw + ci * K, K)], vx, lsem)
                pltpu.async_copy(cy_h.at[pl.ds(wid * epw + ci * K, K)], vy, lsem)
                pltpu.async_copy(cz_h.at[pl.ds(wid * epw + ci * K, K)], vz, lsem)

            def process(ci, st):
                iv, vx, vy, vz, valbuf, lsem = st
                for _ in range(3):
                    pltpu.make_async_copy(cx_h.at[pl.ds(0, K)], vx, lsem).wait()
                pltpu.make_async_copy(idx_h.at[wid, 0], iv, lsem).wait()

                def grp(g, c2):
                    sl = pl.ds(g * 16, 16)
                    row = lane + g * 16
                    plsc.store_scatter(valbuf, [row, lane * 0], vx[sl])
                    plsc.store_scatter(valbuf, [row, lane * 0 + 1], vy[sl])
                    plsc.store_scatter(valbuf, [row, lane * 0 + 2], vz[sl])
                    plsc.store_scatter(valbuf, [row, lane * 0 + 3], one16)
                    return c2

                lax.fori_loop(0, K // 16, grp, 0)
                pltpu.sync_copy(valbuf, acc.at[iv], add=True)

            issue(0, sets[0])
            issue(1, sets[1])

            def pair_body(j, carry):
                c0 = j * 2
                process(c0, sets[0])

                @pl.when(c0 + 2 < nchunks)
                def _():
                    issue(c0 + 2, sets[0])

                process(c0 + 1, sets[1])

                @pl.when(c0 + 3 < nchunks)
                def _():
                    issue(c0 + 3, sets[1])

                return carry

            lax.fori_loop(0, nchunks // 2, pair_body, 0)
            if nchunks % 2:
                process(nchunks - 1, sets[0])

        make_pass(i0_h, o0x_h, o0y_h, o0z_h)
        make_pass(i1_h, p1x_h, p1y_h, p1z_h)
        make_pass(i2_h, o2x_h, o2y_h, o2z_h)
        plsc.subcore_barrier()
        rsl = pl.ds(sid * rows_per_tile, rows_per_tile)
        pltpu.sync_copy(acc.at[rsl, :], stage)
        pltpu.sync_copy(stage, out_h.at[cid, rsl, :])

    return k(o0x, o0y, o0z, p1x, p1y, p1z, o2x, o2y, o2z, i0r, i1r, i2r)


def _finalize(P, pos16):
    n = pos16.shape[0]

    def body(p_ref, pos_ref, out_ref):
        num = p_ref[0] + p_ref[1]
        cnt = num[:, 3:4]
        mean = num / jnp.maximum(cnt, 1.0)
        out_ref[...] = jnp.where(cnt > 0, mean, pos_ref[...])

    return pl.pallas_call(
        body,
        in_specs=[
            pl.BlockSpec((2, n, 16), lambda: (0, 0, 0)),
            pl.BlockSpec((n, 16), lambda: (0, 0)),
        ],
        out_specs=pl.BlockSpec((n, 16), lambda: (0, 0)),
        out_shape=jax.ShapeDtypeStruct((n, 16), jnp.float32),
    )(P, pos16)


def kernel(x, angle_index, node_embedding, pos, angle_types,
           W0, b0, W1, b1, g1, be1, W2, b2, g2, be2, W3, b3, angle_table):
    n = x.shape[0]
    e = angle_index.shape[0]

    i0 = angle_index[:, 0].astype(jnp.int32)
    i1 = angle_index[:, 1].astype(jnp.int32)
    i2 = angle_index[:, 2].astype(jnp.int32)
    xx, xy, xz = x[:, 0], x[:, 1], x[:, 2]
    px, py, pz = pos[:, 0], pos[:, 1], pos[:, 2]

    K = 80
    nchunks = e // (NW * K)
    chunked = lambda a: a.reshape(NW, nchunks, K)
    nt = angle_table.shape[0]
    tt = angle_types.astype(jnp.int32)
    j0 = i0 * nt + tt
    j0r, i1r, i2r = chunked(j0), chunked(i1), chunked(i2)
    i0r = chunked(i0)

    A4, B, C = _precompute(node_embedding, W0, b0[None, :], angle_table)
    s = _sc_embed(A4, B, C, j0r, i1r, i2r)
    (r0x, r0y, r0z, r2x, r2y, r2z,
     q0x, q0y, q0z, q2x, q2y, q2z, p1x, p1y, p1z) = _sc_geom(
        xx, xy, xz, px, py, pz, i0, i1, i2)

    b3p = jnp.concatenate([b3, jnp.zeros((126,), jnp.float32)])[None, :]
    th0, th2 = _mlp(s, W1, b1[None, :], g1[None, :],
                    be1[None, :], W2, b2[None, :], g2[None, :], be2[None, :],
                    W3.T, b3p)

    o0x, o0y, o0z, o2x, o2y, o2z = _rotate(
        th0, th2,
        r0x, r0y, r0z, r2x, r2y, r2z,
        q0x, q0y, q0z, q2x, q2y, q2z, p1x, p1y, p1z)

    npad = ((n + 127) // 128) * 128
    P = _sc_scatter(npad, o0x, o0y, o0z, p1x, p1y, p1z,
                    o2x, o2y, o2z, i0r, i1r, i2r)

    pos16 = jnp.concatenate(
        [pos, jnp.zeros((n, 13), jnp.float32)], axis=1)
    pos16 = jnp.concatenate(
        [pos16, jnp.zeros((npad - n, 16), jnp.float32)], axis=0)
    out16 = _finalize(P, pos16)
    return out16[:n, :3]
